# baseline (speedup 1.0000x reference)
import functools

import jax
import jax.numpy as jnp
from jax import lax
from jax.experimental import pallas as pl
from jax.experimental.pallas import tpu as pltpu
from jax.experimental.pallas import tpu_sc as plsc

_BINS = 256
_NBUF = 4
_NC = 2
_NS = 16
_NW = _NC * _NS
_LANES = 16
_UNROLL = 8
_NCI = 96
_H = 512
_W = 512
_HH = _H // 2
_SPLITS = 3
_RSPLIT = 8 * _NCI // _SPLITS
_RPW = _RSPLIT // _NW
_CHUNK = 16384
_NCH = (_HH * _W) // _CHUNK
_HROWS = _CHUNK // _W
_SPR = _W // _LANES


def _tc_idx(split):
    rg0 = split * (_RSPLIT // 8)
    cpg = _NCI // 8

    def body(x_ref, out_ref):
        v = x_ref[...]
        mn = jnp.min(v, axis=(2, 3), keepdims=True)
        mx = jnp.max(v, axis=(2, 3), keepdims=True)
        same = mn == mx
        mn2 = jnp.where(same, mn - 1.0, mn)
        mx2 = jnp.where(same, mx + 1.0, mx)
        scale = 256.0 / (mx2 - mn2)
        idx = jnp.minimum((v - mn2) * scale, 255.0).astype(jnp.int32)
        lane_pat = (lax.broadcasted_iota(jnp.int32, (1, 1, 1, _W), 3)
                    % _LANES) * _BINS
        idx = idx + lane_pat
        ev = idx[0, :, :_HH, :]
        od = idx[0, :, _HH:, :]
        out_ref[...] = ev | (od << 16)

    return pl.pallas_call(
        body,
        grid=(_RSPLIT // 8,),
        in_specs=[pl.BlockSpec(
            (1, 8, _H, _W),
            lambda i: ((rg0 + i) // cpg, (rg0 + i) % cpg, 0, 0))],
        out_specs=pl.BlockSpec((8, _HH, _W), lambda i: (i, 0, 0)),
        out_shape=jax.ShapeDtypeStruct((_RSPLIT, _HH, _W), jnp.int32),
    )


def _sc_scatter(split):
    def body(idx_hbm, out_hbm, b0, b1, b2, b3, hist, outb,
             s0, s1, s2, s3):
        bufs = [b0, b1, b2, b3]
        sems = [s0, s1, s2, s3]
        wid = lax.axis_index("s") * _NC + lax.axis_index("c")

        lane_base = lax.iota(jnp.int32, _LANES) * _BINS
        ones_v = jnp.full((_LANES,), 1.0, jnp.float32)
        zeros_v = jnp.zeros((_LANES,), jnp.float32)
        lo_mask = jnp.full((_LANES,), 0xFFFF, jnp.int32)

        def fetch(r, c, b):
            return pltpu.async_copy(
                idx_hbm.at[r, pl.ds(c * _HROWS, _HROWS), :],
                bufs[b], sems[b])

        def chunk_wait(b):
            pltpu.make_async_copy(
                idx_hbm.at[0, pl.ds(0, _HROWS), :], bufs[b],
                sems[b]).wait()

        def scatter_chunk(buf):
            def bbody(k):
                v = buf[k // _SPR, pl.ds((k % _SPR) * _LANES, _LANES)]
                lo = v & lo_mask
                hi = lax.shift_right_logical(v, 16)
                plsc.addupdate_scatter(hist, [lo], ones_v)
                plsc.addupdate_scatter(hist, [hi], ones_v)
            plsc.parallel_loop(0, _CHUNK // _LANES, 1,
                               unroll=2 * _UNROLL)(bbody)

        def row_body(i, _):
            r = wid * _RPW + i
            rn = wid * _RPW + jnp.minimum(i + 1, _RPW - 1)

            def zbody(j):
                hist[pl.ds(j * _LANES, _LANES)] = zeros_v
            plsc.parallel_loop(0, _BINS, 1, unroll=4)(zbody)

            for c in range(_NCH):
                b = c % _NBUF
                chunk_wait(b)
                scatter_chunk(bufs[b])
                f = c + _NBUF
                if f < _NCH:
                    fetch(r, f, b)
                else:
                    fetch(rn, f - _NCH, b)

            def fbody(j):
                acc = hist[pl.ds(j * _LANES, _LANES)]
                for lane in range(1, _LANES):
                    acc = acc + hist[pl.ds(lane * _BINS + j * _LANES,
                                           _LANES)]
                outb[i, pl.ds(j * _LANES, _LANES)] = acc
            plsc.parallel_loop(0, _BINS // _LANES, 1, unroll=2)(fbody)
            return 0

        for c in range(_NBUF):
            fetch(wid * _RPW, c, c)
        lax.fori_loop(0, _RPW, row_body, 0)
        for b in range(_NBUF):
            chunk_wait(b)
        pltpu.sync_copy(outb, out_hbm.at[pl.ds(wid * _RPW, _RPW), :])

    return pl.kernel(
        body,
        out_type=jax.ShapeDtypeStruct((_RSPLIT, _BINS), jnp.float32),
        mesh=plsc.VectorSubcoreMesh(core_axis_name="c",
                                    subcore_axis_name="s",
                                    num_cores=_NC, num_subcores=_NS),
        compiler_params=pltpu.CompilerParams(needs_layout_passes=False,
                                             use_tc_tiling_on_sc=True),
        scratch_types=[
            pltpu.VMEM((_HROWS, _W), jnp.int32),
            pltpu.VMEM((_HROWS, _W), jnp.int32),
            pltpu.VMEM((_HROWS, _W), jnp.int32),
            pltpu.VMEM((_HROWS, _W), jnp.int32),
            pltpu.VMEM((_LANES * _BINS,), jnp.float32),
            pltpu.VMEM((_RPW, _BINS), jnp.float32),
            pltpu.SemaphoreType.DMA,
            pltpu.SemaphoreType.DMA,
            pltpu.SemaphoreType.DMA,
            pltpu.SemaphoreType.DMA,
        ],
    )


_tc_stages = [_tc_idx(s) for s in range(_SPLITS)]
_sc_stages = [_sc_scatter(s) for s in range(_SPLITS)]


@jax.jit
def kernel(x):
    b, c, h, w = x.shape
    parts = []
    for s in range(_SPLITS):
        idx = _tc_stages[s](x)
        parts.append(_sc_stages[s](idx))
    hist = jnp.concatenate(parts, axis=0)
    return hist.reshape(b, c, _BINS)


# speedup vs baseline: 1.0013x; 1.0013x over previous
import functools

import jax
import jax.numpy as jnp
from jax import lax
from jax.experimental import pallas as pl
from jax.experimental.pallas import tpu as pltpu
from jax.experimental.pallas import tpu_sc as plsc

_BINS = 256
_NBUF = 4
_NC = 2
_NS = 16
_NW = _NC * _NS
_LANES = 16
_UNROLL = 8
_NCI = 96
_H = 512
_W = 512
_HH = _H // 2
_SPLITS = 3
_RSPLIT = 8 * _NCI // _SPLITS
_RPW = _RSPLIT // _NW
_CHUNK = 16384
_NCH = (_HH * _W) // _CHUNK
_HROWS = _CHUNK // _W
_SPR = _W // _LANES
_HPAD = ((_LANES * (_BINS + 1) + _LANES - 1) // _LANES) * _LANES


def _tc_idx(split):
    rg0 = split * (_RSPLIT // 8)
    cpg = _NCI // 8

    def body(x_ref, out_ref):
        v = x_ref[...]
        mn = jnp.min(v, axis=(2, 3), keepdims=True)
        mx = jnp.max(v, axis=(2, 3), keepdims=True)
        same = mn == mx
        mn2 = jnp.where(same, mn - 1.0, mn)
        mx2 = jnp.where(same, mx + 1.0, mx)
        scale = 256.0 / (mx2 - mn2)
        idx = jnp.minimum((v - mn2) * scale, 255.0).astype(jnp.int32)
        lane_pat = (lax.broadcasted_iota(jnp.int32, (1, 1, 1, _W), 3)
                    % _LANES) * (_BINS + 1)
        idx = idx + lane_pat
        ev = idx[0, :, :_HH, :]
        od = idx[0, :, _HH:, :]
        out_ref[...] = ev | (od << 16)

    return pl.pallas_call(
        body,
        grid=(_RSPLIT // 8,),
        in_specs=[pl.BlockSpec(
            (1, 8, _H, _W),
            lambda i: ((rg0 + i) // cpg, (rg0 + i) % cpg, 0, 0))],
        out_specs=pl.BlockSpec((8, _HH, _W), lambda i: (i, 0, 0)),
        out_shape=jax.ShapeDtypeStruct((_RSPLIT, _HH, _W), jnp.int32),
    )


def _sc_scatter(split):
    def body(idx_hbm, out_hbm, b0, b1, b2, b3, hist, outb,
             s0, s1, s2, s3):
        bufs = [b0, b1, b2, b3]
        sems = [s0, s1, s2, s3]
        wid = lax.axis_index("s") * _NC + lax.axis_index("c")

        lane_base = lax.iota(jnp.int32, _LANES) * _BINS
        ones_v = jnp.full((_LANES,), 1.0, jnp.float32)
        zeros_v = jnp.zeros((_LANES,), jnp.float32)
        lo_mask = jnp.full((_LANES,), 0xFFFF, jnp.int32)

        def fetch(r, c, b):
            return pltpu.async_copy(
                idx_hbm.at[r, pl.ds(c * _HROWS, _HROWS), :],
                bufs[b], sems[b])

        def chunk_wait(b):
            pltpu.make_async_copy(
                idx_hbm.at[0, pl.ds(0, _HROWS), :], bufs[b],
                sems[b]).wait()

        def scatter_chunk(buf):
            def bbody(k):
                v = buf[k // _SPR, pl.ds((k % _SPR) * _LANES, _LANES)]
                lo = v & lo_mask
                hi = lax.shift_right_logical(v, 16)
                plsc.addupdate_scatter(hist, [lo], ones_v)
                plsc.addupdate_scatter(hist, [hi], ones_v)
            plsc.parallel_loop(0, _CHUNK // _LANES, 1,
                               unroll=_UNROLL)(bbody)

        def row_body(i, _):
            r = wid * _RPW + i
            rn = wid * _RPW + jnp.minimum(i + 1, _RPW - 1)

            def zbody(j):
                hist[pl.ds(j * _LANES, _LANES)] = zeros_v
            plsc.parallel_loop(0, _HPAD // _LANES, 1, unroll=4)(zbody)

            for c in range(_NCH):
                b = c % _NBUF
                chunk_wait(b)
                scatter_chunk(bufs[b])
                f = c + _NBUF
                if f < _NCH:
                    fetch(r, f, b)
                else:
                    fetch(rn, f - _NCH, b)

            def fbody(j):
                acc = hist[pl.ds(j * _LANES, _LANES)]
                for lane in range(1, _LANES):
                    acc = acc + hist[pl.ds(lane * (_BINS + 1) +
                                           j * _LANES, _LANES)]
                outb[i, pl.ds(j * _LANES, _LANES)] = acc
            plsc.parallel_loop(0, _BINS // _LANES, 1, unroll=2)(fbody)
            return 0

        for c in range(_NBUF):
            fetch(wid * _RPW, c, c)
        lax.fori_loop(0, _RPW, row_body, 0)
        for b in range(_NBUF):
            chunk_wait(b)
        pltpu.sync_copy(outb, out_hbm.at[pl.ds(wid * _RPW, _RPW), :])

    return pl.kernel(
        body,
        out_type=jax.ShapeDtypeStruct((_RSPLIT, _BINS), jnp.float32),
        mesh=plsc.VectorSubcoreMesh(core_axis_name="c",
                                    subcore_axis_name="s",
                                    num_cores=_NC, num_subcores=_NS),
        compiler_params=pltpu.CompilerParams(needs_layout_passes=False,
                                             use_tc_tiling_on_sc=True),
        scratch_types=[
            pltpu.VMEM((_HROWS, _W), jnp.int32),
            pltpu.VMEM((_HROWS, _W), jnp.int32),
            pltpu.VMEM((_HROWS, _W), jnp.int32),
            pltpu.VMEM((_HROWS, _W), jnp.int32),
            pltpu.VMEM((_HPAD,), jnp.float32),
            pltpu.VMEM((_RPW, _BINS), jnp.float32),
            pltpu.SemaphoreType.DMA,
            pltpu.SemaphoreType.DMA,
            pltpu.SemaphoreType.DMA,
            pltpu.SemaphoreType.DMA,
        ],
    )


_tc_stages = [_tc_idx(s) for s in range(_SPLITS)]
_sc_stages = [_sc_scatter(s) for s in range(_SPLITS)]


@jax.jit
def kernel(x):
    b, c, h, w = x.shape
    parts = []
    for s in range(_SPLITS):
        idx = _tc_stages[s](x)
        parts.append(_sc_stages[s](idx))
    hist = jnp.concatenate(parts, axis=0)
    return hist.reshape(b, c, _BINS)


# speedup vs baseline: 1.0201x; 1.0188x over previous
import functools

import jax
import jax.numpy as jnp
from jax import lax
from jax.experimental import pallas as pl
from jax.experimental.pallas import tpu as pltpu
from jax.experimental.pallas import tpu_sc as plsc

_BINS = 256
_NBUF = 4
_NC = 2
_NS = 16
_NW = _NC * _NS
_LANES = 16
_UNROLL = 8
_NCI = 96
_H = 512
_W = 512
_HH = _H // 2
_SPLIT_SIZES = (128, 192, 224, 224)
_CHUNK = 16384
_NCH = (_HH * _W) // _CHUNK
_HROWS = _CHUNK // _W
_SPR = _W // _LANES


def _tc_idx(row0, nrows):
    rg0 = row0 // 8
    cpg = _NCI // 8

    def body(x_ref, out_ref):
        v = x_ref[...]
        mn = jnp.min(v, axis=(2, 3), keepdims=True)
        mx = jnp.max(v, axis=(2, 3), keepdims=True)
        same = mn == mx
        mn2 = jnp.where(same, mn - 1.0, mn)
        mx2 = jnp.where(same, mx + 1.0, mx)
        scale = 256.0 / (mx2 - mn2)
        idx = jnp.minimum((v - mn2) * scale, 255.0).astype(jnp.int32)
        ev = idx[0, :, :_HH, :]
        od = idx[0, :, _HH:, :]
        out_ref[...] = ev | (od << 16)

    return pl.pallas_call(
        body,
        grid=(nrows // 8,),
        in_specs=[pl.BlockSpec(
            (1, 8, _H, _W),
            lambda i: ((rg0 + i) // cpg, (rg0 + i) % cpg, 0, 0))],
        out_specs=pl.BlockSpec((8, _HH, _W), lambda i: (i, 0, 0)),
        out_shape=jax.ShapeDtypeStruct((nrows, _HH, _W), jnp.int32),
    )


def _sc_scatter(nrows):
    rpw = nrows // _NW

    def body(idx_hbm, out_hbm, b0, b1, b2, b3, hist, outb,
             s0, s1, s2, s3):
        bufs = [b0, b1, b2, b3]
        sems = [s0, s1, s2, s3]
        wid = lax.axis_index("s") * _NC + lax.axis_index("c")

        lane_base = lax.iota(jnp.int32, _LANES) * _BINS
        ones_v = jnp.full((_LANES,), 1.0, jnp.float32)
        zeros_v = jnp.zeros((_LANES,), jnp.float32)
        lo_mask = jnp.full((_LANES,), 0xFFFF, jnp.int32)

        def fetch(r, c, b):
            return pltpu.async_copy(
                idx_hbm.at[r, pl.ds(c * _HROWS, _HROWS), :],
                bufs[b], sems[b])

        def chunk_wait(b):
            pltpu.make_async_copy(
                idx_hbm.at[0, pl.ds(0, _HROWS), :], bufs[b],
                sems[b]).wait()

        def scatter_chunk(buf):
            def bbody(k):
                v = buf[k // _SPR, pl.ds((k % _SPR) * _LANES, _LANES)]
                lo = (v & lo_mask) + lane_base
                hi = lax.shift_right_logical(v, 16) + lane_base
                plsc.addupdate_scatter(hist, [lo], ones_v)
                plsc.addupdate_scatter(hist, [hi], ones_v)
            plsc.parallel_loop(0, _CHUNK // _LANES, 1,
                               unroll=_UNROLL)(bbody)

        def row_body(i, _):
            r = wid * rpw + i
            rn = wid * rpw + jnp.minimum(i + 1, rpw - 1)

            def zbody(j):
                hist[pl.ds(j * _LANES, _LANES)] = zeros_v
            plsc.parallel_loop(0, _BINS, 1, unroll=4)(zbody)

            for c in range(_NCH):
                b = c % _NBUF
                chunk_wait(b)
                scatter_chunk(bufs[b])
                f = c + _NBUF
                if f < _NCH:
                    fetch(r, f, b)
                else:
                    fetch(rn, f - _NCH, b)

            def fbody(j):
                acc = hist[pl.ds(j * _LANES, _LANES)]
                for lane in range(1, _LANES):
                    acc = acc + hist[pl.ds(lane * _BINS + j * _LANES,
                                           _LANES)]
                outb[i, pl.ds(j * _LANES, _LANES)] = acc
            plsc.parallel_loop(0, _BINS // _LANES, 1, unroll=2)(fbody)
            return 0

        for c in range(_NBUF):
            fetch(wid * rpw, c, c)
        lax.fori_loop(0, rpw, row_body, 0)
        for b in range(_NBUF):
            chunk_wait(b)
        pltpu.sync_copy(outb, out_hbm.at[wid])

    return pl.kernel(
        body,
        out_type=jax.ShapeDtypeStruct((_NW, rpw, _BINS), jnp.float32),
        mesh=plsc.VectorSubcoreMesh(core_axis_name="c",
                                    subcore_axis_name="s",
                                    num_cores=_NC, num_subcores=_NS),
        compiler_params=pltpu.CompilerParams(needs_layout_passes=False,
                                             use_tc_tiling_on_sc=True),
        scratch_types=[
            pltpu.VMEM((_HROWS, _W), jnp.int32),
            pltpu.VMEM((_HROWS, _W), jnp.int32),
            pltpu.VMEM((_HROWS, _W), jnp.int32),
            pltpu.VMEM((_HROWS, _W), jnp.int32),
            pltpu.VMEM((_LANES * _BINS,), jnp.float32),
            pltpu.VMEM((rpw, _BINS), jnp.float32),
            pltpu.SemaphoreType.DMA,
            pltpu.SemaphoreType.DMA,
            pltpu.SemaphoreType.DMA,
            pltpu.SemaphoreType.DMA,
        ],
    )


_row_starts = [sum(_SPLIT_SIZES[:s]) for s in range(len(_SPLIT_SIZES))]
_tc_stages = [_tc_idx(r0, n) for r0, n in zip(_row_starts, _SPLIT_SIZES)]
_sc_stages = [_sc_scatter(n) for n in _SPLIT_SIZES]


@jax.jit
def kernel(x):
    b, c, h, w = x.shape
    parts = []
    for s, n in enumerate(_SPLIT_SIZES):
        idx = _tc_stages[s](x)
        parts.append(_sc_stages[s](idx).reshape(n, _BINS))
    hist = jnp.concatenate(parts, axis=0)
    return hist.reshape(b, c, _BINS)


# speedup vs baseline: 1.0205x; 1.0004x over previous
import jax
import jax.numpy as jnp
from jax import lax
from jax.experimental import pallas as pl
from jax.experimental.pallas import tpu as pltpu
from jax.experimental.pallas import tpu_sc as plsc

_BINS = 256
_NBUF = 4
_NC = 2
_NS = 16
_NW = _NC * _NS
_LANES = 16
_UNROLL = 8
_NCI = 96
_H = 512
_W = 512
_HH = _H // 2
_SPLIT_SIZES = (128, 192, 224, 224)
_CHUNK = 16384
_NCH = (_HH * _W) // _CHUNK
_HROWS = _CHUNK // _W
_SPR = _W // _LANES


def _tc_idx(row0, nrows):
    rg0 = row0 // 8
    cpg = _NCI // 8

    def body(x_ref, out_ref):
        v = x_ref[...]
        mn = jnp.min(v, axis=(2, 3), keepdims=True)
        mx = jnp.max(v, axis=(2, 3), keepdims=True)
        same = mn == mx
        mn2 = jnp.where(same, mn - 1.0, mn)
        mx2 = jnp.where(same, mx + 1.0, mx)
        scale = 256.0 / (mx2 - mn2)
        idx = jnp.minimum((v - mn2) * scale, 255.0).astype(jnp.int32)
        ev = idx[0, :, :_HH, :]
        od = idx[0, :, _HH:, :]
        out_ref[...] = ev | (od << 16)

    return pl.pallas_call(
        body,
        grid=(nrows // 8,),
        in_specs=[pl.BlockSpec(
            (1, 8, _H, _W),
            lambda i: ((rg0 + i) // cpg, (rg0 + i) % cpg, 0, 0))],
        out_specs=pl.BlockSpec((8, _HH, _W), lambda i: (i, 0, 0)),
        out_shape=jax.ShapeDtypeStruct((nrows, _HH, _W), jnp.int32),
    )


def _sc_scatter(nrows):
    rpw = nrows // _NW

    def body(idx_hbm, out_hbm, b0, b1, b2, b3, hist, outb,
             s0, s1, s2, s3):
        bufs = [b0, b1, b2, b3]
        sems = [s0, s1, s2, s3]
        wid = lax.axis_index("s") * _NC + lax.axis_index("c")

        lane_base = lax.iota(jnp.int32, _LANES) * _BINS
        ones_v = jnp.full((_LANES,), 1.0, jnp.float32)
        zeros_v = jnp.zeros((_LANES,), jnp.float32)
        lo_mask = jnp.full((_LANES,), 0xFFFF, jnp.int32)

        def fetch(r, c, b):
            return pltpu.async_copy(
                idx_hbm.at[r, pl.ds(c * _HROWS, _HROWS), :],
                bufs[b], sems[b])

        def chunk_wait(b):
            pltpu.make_async_copy(
                idx_hbm.at[0, pl.ds(0, _HROWS), :], bufs[b],
                sems[b]).wait()

        def scatter_chunk(buf):
            def bbody(k):
                v = buf[k // _SPR, pl.ds((k % _SPR) * _LANES, _LANES)]
                lo = (v & lo_mask) + lane_base
                hi = lax.shift_right_logical(v, 16) + lane_base
                plsc.addupdate_scatter(hist, [lo], ones_v)
                plsc.addupdate_scatter(hist, [hi], ones_v)
            plsc.parallel_loop(0, _CHUNK // _LANES, 1,
                               unroll=_UNROLL)(bbody)

        def row_body(i, _):
            r = wid * rpw + i
            rn = wid * rpw + jnp.minimum(i + 1, rpw - 1)

            def zbody(j):
                hist[pl.ds(j * _LANES, _LANES)] = zeros_v
            plsc.parallel_loop(0, _BINS, 1, unroll=4)(zbody)

            for c in range(_NCH):
                b = c % _NBUF
                chunk_wait(b)
                scatter_chunk(bufs[b])
                f = c + _NBUF
                if f < _NCH:
                    fetch(r, f, b)
                else:
                    fetch(rn, f - _NCH, b)

            def fbody(j):
                acc = hist[pl.ds(j * _LANES, _LANES)]
                for lane in range(1, _LANES):
                    acc = acc + hist[pl.ds(lane * _BINS + j * _LANES,
                                           _LANES)]
                outb[i, pl.ds(j * _LANES, _LANES)] = acc
            plsc.parallel_loop(0, _BINS // _LANES, 1, unroll=2)(fbody)
            return 0

        for c in range(_NBUF):
            fetch(wid * rpw, c, c)
        lax.fori_loop(0, rpw, row_body, 0)
        for b in range(_NBUF):
            chunk_wait(b)
        pltpu.sync_copy(outb, out_hbm.at[wid])

    return pl.kernel(
        body,
        out_type=jax.ShapeDtypeStruct((_NW, rpw, _BINS), jnp.float32),
        mesh=plsc.VectorSubcoreMesh(core_axis_name="c",
                                    subcore_axis_name="s",
                                    num_cores=_NC, num_subcores=_NS),
        compiler_params=pltpu.CompilerParams(needs_layout_passes=False,
                                             use_tc_tiling_on_sc=True),
        scratch_types=[
            pltpu.VMEM((_HROWS, _W), jnp.int32),
            pltpu.VMEM((_HROWS, _W), jnp.int32),
            pltpu.VMEM((_HROWS, _W), jnp.int32),
            pltpu.VMEM((_HROWS, _W), jnp.int32),
            pltpu.VMEM((_LANES * _BINS,), jnp.float32),
            pltpu.VMEM((rpw, _BINS), jnp.float32),
            pltpu.SemaphoreType.DMA,
            pltpu.SemaphoreType.DMA,
            pltpu.SemaphoreType.DMA,
            pltpu.SemaphoreType.DMA,
        ],
    )


_row_starts = [sum(_SPLIT_SIZES[:s]) for s in range(len(_SPLIT_SIZES))]
_tc_stages = [_tc_idx(r0, n) for r0, n in zip(_row_starts, _SPLIT_SIZES)]
_sc_stages = [_sc_scatter(n) for n in _SPLIT_SIZES]


@jax.jit
def kernel(x):
    b, c, h, w = x.shape
    parts = []
    for s, n in enumerate(_SPLIT_SIZES):
        idx = _tc_stages[s](x)
        parts.append(_sc_stages[s](idx).reshape(n, _BINS))
    hist = jnp.concatenate(parts, axis=0)
    return hist.reshape(b, c, _BINS)
